# baseline (speedup 1.0000x reference)
import functools

import jax
import jax.numpy as jnp
from jax import lax
from jax.experimental import pallas as pl
from jax.experimental.pallas import tpu as pltpu
from jax.experimental.pallas import tpu_sc as plsc

D_MODEL = 1024
N_ROWS = 32768
NC = 2
NS = 16
NW = NC * NS
PER_W = N_ROWS // NW
GRP = 16


def _sc_gather(table, idx):
    mesh = plsc.VectorSubcoreMesh(core_axis_name="c", subcore_axis_name="s")

    @functools.partial(
        pl.kernel,
        mesh=mesh,
        out_type=jax.ShapeDtypeStruct((N_ROWS, D_MODEL), jnp.float32),
        scratch_types=[
            pltpu.VMEM((PER_W,), jnp.int32),
            pltpu.SemaphoreType.DMA,
        ],
    )
    def k(table_hbm, idx_hbm, out_hbm, idx_v, sem):
        wid = lax.axis_index("s") * NC + lax.axis_index("c")
        base = wid * PER_W
        pltpu.sync_copy(idx_hbm.at[pl.ds(base, PER_W)], idx_v)

        def body(g, _):
            rows = idx_v[pl.ds(g * GRP, GRP)]
            for u in range(GRP):
                r = rows[u]
                pltpu.async_copy(
                    table_hbm.at[pl.ds(r, 1)],
                    out_hbm.at[pl.ds(base + g * GRP + u, 1)], sem)
            for u in range(GRP):
                pltpu.make_async_copy(
                    table_hbm.at[pl.ds(0, 1)],
                    out_hbm.at[pl.ds(base + g * GRP + u, 1)], sem).wait()
            return 0

        lax.fori_loop(0, PER_W // GRP, body, 0)

    return k(table, idx)


def kernel(positions, pe):
    idx = positions.reshape(-1).astype(jnp.int32)
    table = pe.reshape(pe.shape[-2], pe.shape[-1])
    out = _sc_gather(table, idx)
    return out.reshape(positions.shape[0], positions.shape[1], D_MODEL)


# speedup vs baseline: 36.2851x; 36.2851x over previous
import functools

import jax
import jax.numpy as jnp
from jax import lax
from jax.experimental import pallas as pl
from jax.experimental.pallas import tpu as pltpu
from jax.experimental.pallas import tpu_sc as plsc

D_MODEL = 1024
N_ROWS = 32768
NC = 2
NS = 16
NW = NC * NS
PER_W = N_ROWS // NW
CHUNK = 16
NCHUNK = PER_W // CHUNK
NBUF = 4
LOOK = NBUF - 1


def _sc_gather(table, idx):
    mesh = plsc.VectorSubcoreMesh(core_axis_name="c", subcore_axis_name="s")

    @functools.partial(
        pl.kernel,
        mesh=mesh,
        out_type=jax.ShapeDtypeStruct((N_ROWS, D_MODEL), jnp.float32),
        scratch_types=[
            pltpu.VMEM((PER_W,), jnp.int32),
            pltpu.VMEM((NBUF, CHUNK, D_MODEL), jnp.float32),
            pltpu.SemaphoreType.DMA((NBUF,)),
            pltpu.SemaphoreType.DMA((NBUF,)),
        ],
    )
    def k(table_hbm, idx_hbm, out_hbm, idx_v, bufs, gsem, psem):
        wid = lax.axis_index("s") * NC + lax.axis_index("c")
        base = wid * PER_W
        pltpu.sync_copy(idx_hbm.at[pl.ds(base, PER_W)], idx_v)

        def g_start(i, b):
            pltpu.async_copy(
                table_hbm.at[idx_v.at[pl.ds(i * CHUNK, CHUNK)]],
                bufs.at[b], gsem.at[b])

        def g_wait(i, b):
            pltpu.make_async_copy(
                table_hbm.at[idx_v.at[pl.ds(i * CHUNK, CHUNK)]],
                bufs.at[b], gsem.at[b]).wait()

        def p_start(i, b):
            pltpu.async_copy(
                bufs.at[b], out_hbm.at[pl.ds(base + i * CHUNK, CHUNK)],
                psem.at[b])

        def p_wait(i, b):
            pltpu.make_async_copy(
                bufs.at[b], out_hbm.at[pl.ds(base + i * CHUNK, CHUNK)],
                psem.at[b]).wait()

        def step(i, b, do_pwait, do_refill):
            g_wait(i, b)
            p_start(i, b)
            if do_pwait:
                p_wait(i - 1, (i - 1) % NBUF)
            if do_refill:
                g_start(i + LOOK, (i + LOOK) % NBUF)

        for j in range(LOOK):
            g_start(j, j)

        for i in range(NBUF):
            step(i, i, i >= 1, i + LOOK < NCHUNK)

        assert NCHUNK % NBUF == 0 and NCHUNK >= 3 * NBUF

        def body(g, _):
            for u in range(NBUF):
                i = NBUF * (g + 1) + u
                step(i, u, True, True)
            return 0

        lax.fori_loop(0, (NCHUNK - 2 * NBUF) // NBUF, body, 0)

        for i in range(NCHUNK - NBUF, NCHUNK):
            step(i, i % NBUF, True, i + LOOK < NCHUNK)
        p_wait(NCHUNK - 1, (NCHUNK - 1) % NBUF)

    return k(table, idx)


def kernel(positions, pe):
    idx = positions.reshape(-1).astype(jnp.int32)
    table = pe.reshape(pe.shape[-2], pe.shape[-1])
    out = _sc_gather(table, idx)
    return out.reshape(positions.shape[0], positions.shape[1], D_MODEL)


# speedup vs baseline: 36.4799x; 1.0054x over previous
import functools

import jax
import jax.numpy as jnp
from jax import lax
from jax.experimental import pallas as pl
from jax.experimental.pallas import tpu as pltpu
from jax.experimental.pallas import tpu_sc as plsc

D_MODEL = 1024
N_ROWS = 32768
NC = 2
NS = 16
NW = NC * NS
PER_W = N_ROWS // NW
CHUNK = 8
NCHUNK = PER_W // CHUNK
NBUF = 8
LOOK = NBUF - 1


def _sc_gather(table, idx):
    mesh = plsc.VectorSubcoreMesh(core_axis_name="c", subcore_axis_name="s")

    @functools.partial(
        pl.kernel,
        mesh=mesh,
        out_type=jax.ShapeDtypeStruct((N_ROWS, D_MODEL), jnp.float32),
        scratch_types=[
            pltpu.VMEM((PER_W,), jnp.int32),
            pltpu.VMEM((NBUF, CHUNK, D_MODEL), jnp.float32),
            pltpu.SemaphoreType.DMA((NBUF,)),
            pltpu.SemaphoreType.DMA((NBUF,)),
        ],
    )
    def k(table_hbm, idx_hbm, out_hbm, idx_v, bufs, gsem, psem):
        wid = lax.axis_index("s") * NC + lax.axis_index("c")
        base = wid * PER_W
        pltpu.sync_copy(idx_hbm.at[pl.ds(base, PER_W)], idx_v)

        def g_start(i, b):
            pltpu.async_copy(
                table_hbm.at[idx_v.at[pl.ds(i * CHUNK, CHUNK)]],
                bufs.at[b], gsem.at[b])

        def g_wait(i, b):
            pltpu.make_async_copy(
                table_hbm.at[idx_v.at[pl.ds(i * CHUNK, CHUNK)]],
                bufs.at[b], gsem.at[b]).wait()

        def p_start(i, b):
            pltpu.async_copy(
                bufs.at[b], out_hbm.at[pl.ds(base + i * CHUNK, CHUNK)],
                psem.at[b])

        def p_wait(i, b):
            pltpu.make_async_copy(
                bufs.at[b], out_hbm.at[pl.ds(base + i * CHUNK, CHUNK)],
                psem.at[b]).wait()

        def step(i, b, do_pwait, do_refill):
            g_wait(i, b)
            p_start(i, b)
            if do_pwait:
                p_wait(i - 1, (i - 1) % NBUF)
            if do_refill:
                g_start(i + LOOK, (i + LOOK) % NBUF)

        for j in range(LOOK):
            g_start(j, j)

        for i in range(NBUF):
            step(i, i, i >= 1, i + LOOK < NCHUNK)

        assert NCHUNK % NBUF == 0 and NCHUNK >= 3 * NBUF

        def body(g, _):
            for u in range(NBUF):
                i = NBUF * (g + 1) + u
                step(i, u, True, True)
            return 0

        lax.fori_loop(0, (NCHUNK - 2 * NBUF) // NBUF, body, 0)

        for i in range(NCHUNK - NBUF, NCHUNK):
            step(i, i % NBUF, True, i + LOOK < NCHUNK)
        p_wait(NCHUNK - 1, (NCHUNK - 1) % NBUF)

    return k(table, idx)


def kernel(positions, pe):
    idx = positions.reshape(-1).astype(jnp.int32)
    table = pe.reshape(pe.shape[-2], pe.shape[-1])
    out = _sc_gather(table, idx)
    return out.reshape(positions.shape[0], positions.shape[1], D_MODEL)
